# baseline (speedup 1.0000x reference)
import jax
import jax.numpy as jnp
from jax import lax
from jax.experimental import pallas as pl

N = 32768
D_IN, D_MODEL, E, D_OUT = 4, 16, 8, 4
BLOCK = 32768

_DN_00 = (((0,), (0,)), ((), ()))
_DN_01 = (((0,), (1,)), ((), ()))


def _moe_kernel(x_ref, Wemb_ref, bemb_ref, Wg_ref, We_ref, beAll_ref,
                Wr_ref, br_ref, Wc_ref, bc_ref, Wp_ref, bp_ref, out_ref):
    f32 = jnp.float32
    x = x_ref[...]
    hT = lax.dot_general(Wemb_ref[...], x, _DN_01,
                         preferred_element_type=f32) + bemb_ref[...]
    logits = lax.dot_general(Wg_ref[...], hT, _DN_00,
                             preferred_element_type=f32)
    m = jnp.max(logits, axis=0, keepdims=True)
    ex = jnp.exp(logits - m)
    denom = jnp.sum(ex, axis=0, keepdims=True)
    gate = jnp.max(ex, axis=0, keepdims=True) / denom
    iota = lax.broadcasted_iota(jnp.int32, logits.shape, 0)
    ismax = logits >= m
    idx = jnp.min(jnp.where(ismax, iota, E), axis=0, keepdims=True)
    onehot = (iota == idx).astype(f32)

    WeAllT = jnp.transpose(We_ref[...], (0, 2, 1)).reshape(
        E * D_MODEL, D_MODEL)
    EO = jnp.dot(WeAllT, hT, preferred_element_type=f32)
    EO = EO + beAll_ref[...]
    acc = jnp.zeros(hT.shape, f32)
    for e in range(E):
        acc = acc + EO[e * D_MODEL:(e + 1) * D_MODEL, :] * onehot[e:e + 1, :]
    moe = gate * acc

    res = lax.dot_general(Wr_ref[...], hT, _DN_00,
                          preferred_element_type=f32) + br_ref[...]
    c = lax.dot_general(Wc_ref[...], hT, _DN_00,
                        preferred_element_type=f32) + bc_ref[...]
    cm = jnp.max(c, axis=0, keepdims=True)
    cex = jnp.exp(c - cm)
    coef = cex / jnp.sum(cex, axis=0, keepdims=True)

    comb = moe * coef[0:1, :] + res * coef[1:2, :]
    out_ref[...] = (lax.dot_general(comb, Wp_ref[...], _DN_00,
                                    preferred_element_type=f32)
                    + bp_ref[...])


@jax.jit
def kernel(x, W_embed, b_embed, Wg, We, be, Wr, br, Wc, bc, Wp, bp):
    grid = (N // BLOCK,)

    def full(shape):
        return pl.BlockSpec(shape, lambda i: tuple(0 for _ in shape))

    out = pl.pallas_call(
        _moe_kernel,
        grid=grid,
        in_specs=[
            pl.BlockSpec((BLOCK, D_IN), lambda i: (i, 0)),
            full((D_IN, D_MODEL)),
            full((D_MODEL, 1)),
            full((D_MODEL, E)),
            full((E, D_MODEL, D_MODEL)),
            full((E * D_MODEL, 1)),
            full((D_MODEL, D_MODEL)),
            full((D_MODEL, 1)),
            full((D_MODEL, 2)),
            full((2, 1)),
            full((D_MODEL, D_OUT)),
            full((1, D_OUT)),
        ],
        out_specs=pl.BlockSpec((BLOCK, D_OUT), lambda i: (i, 0)),
        out_shape=jax.ShapeDtypeStruct((N, D_OUT), jnp.float32),
    )(x,
      W_embed, b_embed.reshape(-1, 1),
      Wg,
      We,
      be.reshape(-1, 1),
      Wr, br.reshape(-1, 1),
      Wc, bc.reshape(-1, 1),
      Wp, bp.reshape(1, -1))
    return out


# speedup vs baseline: 1.0630x; 1.0630x over previous
import jax
import jax.numpy as jnp
from jax import lax
from jax.experimental import pallas as pl

N = 32768
D_IN, D_MODEL, E, D_OUT = 4, 16, 8, 4
BLOCK = 4096

_DN_00 = (((0,), (0,)), ((), ()))
_DN_01 = (((0,), (1,)), ((), ()))


def _moe_kernel(x_ref, Wemb_ref, bemb_ref, Wg_ref, We_ref, beAll_ref,
                Wr_ref, br_ref, Wc_ref, bc_ref, Wp_ref, bp_ref, out_ref):
    f32 = jnp.float32
    x = x_ref[...]
    hT = lax.dot_general(Wemb_ref[...], x, _DN_01,
                         preferred_element_type=f32) + bemb_ref[...]
    logits = lax.dot_general(Wg_ref[...], hT, _DN_00,
                             preferred_element_type=f32)
    m = jnp.max(logits, axis=0, keepdims=True)
    ex = jnp.exp(logits - m)
    denom = jnp.sum(ex, axis=0, keepdims=True)
    gate = jnp.max(ex, axis=0, keepdims=True) / denom
    iota = lax.broadcasted_iota(jnp.int32, logits.shape, 0)
    ismax = logits >= m
    idx = jnp.min(jnp.where(ismax, iota, E), axis=0, keepdims=True)
    onehot = (iota == idx).astype(f32)

    WeAllT = jnp.transpose(We_ref[...], (0, 2, 1)).reshape(
        E * D_MODEL, D_MODEL)
    EO = jnp.dot(WeAllT, hT, preferred_element_type=f32)
    EO = EO + beAll_ref[...]
    acc = jnp.zeros(hT.shape, f32)
    for e in range(E):
        acc = acc + EO[e * D_MODEL:(e + 1) * D_MODEL, :] * onehot[e:e + 1, :]
    moe = gate * acc

    res = lax.dot_general(Wr_ref[...], hT, _DN_00,
                          preferred_element_type=f32) + br_ref[...]
    c = lax.dot_general(Wc_ref[...], hT, _DN_00,
                        preferred_element_type=f32) + bc_ref[...]
    cm = jnp.max(c, axis=0, keepdims=True)
    cex = jnp.exp(c - cm)
    coef = cex / jnp.sum(cex, axis=0, keepdims=True)

    comb = moe * coef[0:1, :] + res * coef[1:2, :]
    out_ref[...] = (lax.dot_general(comb, Wp_ref[...], _DN_00,
                                    preferred_element_type=f32)
                    + bp_ref[...])


@jax.jit
def kernel(x, W_embed, b_embed, Wg, We, be, Wr, br, Wc, bc, Wp, bp):
    grid = (N // BLOCK,)

    def full(shape):
        return pl.BlockSpec(shape, lambda i: tuple(0 for _ in shape))

    out = pl.pallas_call(
        _moe_kernel,
        grid=grid,
        in_specs=[
            pl.BlockSpec((BLOCK, D_IN), lambda i: (i, 0)),
            full((D_IN, D_MODEL)),
            full((D_MODEL, 1)),
            full((D_MODEL, E)),
            full((E, D_MODEL, D_MODEL)),
            full((E * D_MODEL, 1)),
            full((D_MODEL, D_MODEL)),
            full((D_MODEL, 1)),
            full((D_MODEL, 2)),
            full((2, 1)),
            full((D_MODEL, D_OUT)),
            full((1, D_OUT)),
        ],
        out_specs=pl.BlockSpec((BLOCK, D_OUT), lambda i: (i, 0)),
        out_shape=jax.ShapeDtypeStruct((N, D_OUT), jnp.float32),
    )(x,
      W_embed, b_embed.reshape(-1, 1),
      Wg,
      We,
      be.reshape(-1, 1),
      Wr, br.reshape(-1, 1),
      Wc, bc.reshape(-1, 1),
      Wp, bp.reshape(1, -1))
    return out


# speedup vs baseline: 1.0887x; 1.0242x over previous
import jax
import jax.numpy as jnp
from jax import lax
from jax.experimental import pallas as pl

N = 32768
D_IN, D_MODEL, E, D_OUT = 4, 16, 8, 4
BLOCK = 16384

_DN_00 = (((0,), (0,)), ((), ()))
_DN_01 = (((0,), (1,)), ((), ()))


def _moe_kernel(x_ref, Wemb_ref, bemb_ref, Wg_ref, We_ref, beAll_ref,
                Wr_ref, br_ref, Wc_ref, bc_ref, Wp_ref, bp_ref, out_ref):
    f32 = jnp.float32
    x = x_ref[...]
    hT = lax.dot_general(Wemb_ref[...], x, _DN_01,
                         preferred_element_type=f32) + bemb_ref[...]
    logits = lax.dot_general(Wg_ref[...], hT, _DN_00,
                             preferred_element_type=f32)
    m = jnp.max(logits, axis=0, keepdims=True)
    ex = jnp.exp(logits - m)
    denom = jnp.sum(ex, axis=0, keepdims=True)
    gate = jnp.max(ex, axis=0, keepdims=True) / denom
    iota = lax.broadcasted_iota(jnp.int32, logits.shape, 0)
    ismax = logits >= m
    idx = jnp.min(jnp.where(ismax, iota, E), axis=0, keepdims=True)
    onehot = (iota == idx).astype(f32)

    WeAllT = jnp.transpose(We_ref[...], (0, 2, 1)).reshape(
        E * D_MODEL, D_MODEL)
    EO = jnp.dot(WeAllT, hT, preferred_element_type=f32)
    EO = EO + beAll_ref[...]
    acc = jnp.zeros(hT.shape, f32)
    for e in range(E):
        acc = acc + EO[e * D_MODEL:(e + 1) * D_MODEL, :] * onehot[e:e + 1, :]
    moe = gate * acc

    res = lax.dot_general(Wr_ref[...], hT, _DN_00,
                          preferred_element_type=f32) + br_ref[...]
    c = lax.dot_general(Wc_ref[...], hT, _DN_00,
                        preferred_element_type=f32) + bc_ref[...]
    cm = jnp.max(c, axis=0, keepdims=True)
    cex = jnp.exp(c - cm)
    coef = cex / jnp.sum(cex, axis=0, keepdims=True)

    comb = moe * coef[0:1, :] + res * coef[1:2, :]
    out_ref[...] = (lax.dot_general(comb, Wp_ref[...], _DN_00,
                                    preferred_element_type=f32)
                    + bp_ref[...])


@jax.jit
def kernel(x, W_embed, b_embed, Wg, We, be, Wr, br, Wc, bc, Wp, bp):
    grid = (N // BLOCK,)

    def full(shape):
        return pl.BlockSpec(shape, lambda i: tuple(0 for _ in shape))

    out = pl.pallas_call(
        _moe_kernel,
        grid=grid,
        in_specs=[
            pl.BlockSpec((BLOCK, D_IN), lambda i: (i, 0)),
            full((D_IN, D_MODEL)),
            full((D_MODEL, 1)),
            full((D_MODEL, E)),
            full((E, D_MODEL, D_MODEL)),
            full((E * D_MODEL, 1)),
            full((D_MODEL, D_MODEL)),
            full((D_MODEL, 1)),
            full((D_MODEL, 2)),
            full((2, 1)),
            full((D_MODEL, D_OUT)),
            full((1, D_OUT)),
        ],
        out_specs=pl.BlockSpec((BLOCK, D_OUT), lambda i: (i, 0)),
        out_shape=jax.ShapeDtypeStruct((N, D_OUT), jnp.float32),
    )(x,
      W_embed, b_embed.reshape(-1, 1),
      Wg,
      We,
      be.reshape(-1, 1),
      Wr, br.reshape(-1, 1),
      Wc, bc.reshape(-1, 1),
      Wp, bp.reshape(1, -1))
    return out


# speedup vs baseline: 1.1581x; 1.0637x over previous
import jax
import jax.numpy as jnp
from jax import lax
from jax.experimental import pallas as pl
from jax.experimental.pallas import tpu as pltpu

N = 32768
D_IN, D_MODEL, E, D_OUT = 4, 16, 8, 4
BLOCK = 8192

_DN_00 = (((0,), (0,)), ((), ()))
_DN_01 = (((0,), (1,)), ((), ()))


def _moe_kernel(x_ref, Wemb_ref, bembC_ref, Wg_ref, We_ref, beC_ref,
                Wr_ref, brC_ref, Wc_ref, bcC_ref, Wp_ref, bp_ref, out_ref,
                WF_s, bias_s):
    f32 = jnp.float32
    dot = lambda a, b: jnp.dot(a, b, preferred_element_type=f32)
    dg = lambda a, b, dn: lax.dot_general(a, b, dn, preferred_element_type=f32)

    @pl.when(pl.program_id(0) == 0)
    def _fold():
        Wemb = Wemb_ref[...]
        Wp = Wp_ref[...]
        bembC = bembC_ref[...]
        WgF = dot(Wemb, Wg_ref[...])
        bgC = dg(Wg_ref[...], bembC, _DN_00)
        WcF = dot(Wemb, Wc_ref[...])
        bcF = dg(Wc_ref[...], bembC, _DN_00) + bcC_ref[...]
        WrP = dot(Wr_ref[...], Wp)
        WrF = dot(Wemb, WrP)
        brF = dg(WrP, bembC, _DN_00) + dg(Wp, brC_ref[...], _DN_00)
        eW, eB = [], []
        for e in range(E):
            WeP = dot(We_ref[e], Wp)
            eW.append(dot(Wemb, WeP))
            eB.append(dg(WeP, bembC, _DN_00)
                      + dg(Wp, beC_ref[16 * e:16 * (e + 1), :], _DN_00))
        WF_s[...] = jnp.concatenate(eW + [WgF, WrF, WcF], axis=1)
        bias_s[...] = jnp.concatenate(eB + [bgC, brF, bcF], axis=0)

    WF = WF_s[...]
    biasC = bias_s[...]

    x = x_ref[...]
    A = dg(WF, x, _DN_01) + biasC
    logits = A[32:40, :]
    m = jnp.max(logits, axis=0, keepdims=True)
    ex = jnp.exp(logits - m)
    denom = jnp.sum(ex, axis=0, keepdims=True)
    gate = jnp.max(ex, axis=0, keepdims=True) / denom
    iota = lax.broadcasted_iota(jnp.int32, logits.shape, 0)
    idx = jnp.min(jnp.where(logits >= m, iota, E), axis=0, keepdims=True)

    iota32 = lax.broadcasted_iota(jnp.int32, (E * D_OUT, x.shape[0]), 0)
    mask32 = (lax.div(iota32, 4) == idx).astype(f32)
    masked = A[0:32, :] * mask32
    s1 = masked[0:16, :] + masked[16:32, :]
    s2 = s1[0:8, :] + s1[8:16, :]
    acc = s2[0:4, :] + s2[4:8, :]

    c = A[44:46, :]
    cm = jnp.max(c, axis=0, keepdims=True)
    cex = jnp.exp(c - cm)
    coef = cex / jnp.sum(cex, axis=0, keepdims=True)

    outF = gate * coef[0:1, :] * acc + coef[1:2, :] * A[40:44, :]
    out_ref[...] = dg(outF, jnp.eye(4, dtype=f32), _DN_00) + bp_ref[...]


@jax.jit
def kernel(x, W_embed, b_embed, Wg, We, be, Wr, br, Wc, bc, Wp, bp):
    grid = (N // BLOCK,)

    def full(shape):
        return pl.BlockSpec(shape, lambda i: tuple(0 for _ in shape))

    out = pl.pallas_call(
        _moe_kernel,
        grid=grid,
        in_specs=[
            pl.BlockSpec((BLOCK, D_IN), lambda i: (i, 0)),
            full((D_IN, D_MODEL)),
            full((D_MODEL, 1)),
            full((D_MODEL, E)),
            full((E, D_MODEL, D_MODEL)),
            full((E * D_MODEL, 1)),
            full((D_MODEL, D_MODEL)),
            full((D_MODEL, 1)),
            full((D_MODEL, 2)),
            full((2, 1)),
            full((D_MODEL, D_OUT)),
            full((1, D_OUT)),
        ],
        out_specs=pl.BlockSpec((BLOCK, D_OUT), lambda i: (i, 0)),
        out_shape=jax.ShapeDtypeStruct((N, D_OUT), jnp.float32),
        scratch_shapes=[pltpu.VMEM((D_IN, 46), jnp.float32),
                        pltpu.VMEM((46, 1), jnp.float32)],
    )(x,
      W_embed, b_embed.reshape(-1, 1),
      Wg,
      We,
      be.reshape(-1, 1),
      Wr, br.reshape(-1, 1),
      Wc, bc.reshape(-1, 1),
      Wp, bp.reshape(1, -1))
    return out
